# baseline (speedup 1.0000x reference)
import jax
import jax.numpy as jnp
from jax.experimental import pallas as pl
from jax.experimental.pallas import tpu as pltpu

G = 16
K = 3
S = 2
C = 3
IMG = 224
H_G = 128
STD = 0.17
HIDDEN = 256
NCLS = 1000
SIZES = tuple(G * (S ** i) for i in range(K))


def _select_pool_rows(off, f, size):
    g = jax.lax.broadcasted_iota(jnp.int32, (G, size), 0)
    u = jax.lax.broadcasted_iota(jnp.int32, (G, size), 1)
    q = u - off - g * f
    sel = jnp.logical_and(q >= 0, q < f)
    return jnp.where(sel, jnp.float32(1.0 / f), jnp.float32(0.0))


def _select_pool_cols(d1, f):
    j = jax.lax.broadcasted_iota(jnp.int32, (IMG * C, G * C), 0)
    t = jax.lax.broadcasted_iota(jnp.int32, (IMG * C, G * C), 1)
    w = j // C
    cj = j - w * C
    g2 = t // C
    ct = t - g2 * C
    q = w - d1 - g2 * f
    sel = jnp.logical_and(jnp.logical_and(q >= 0, q < f), cj == ct)
    return jnp.where(sel, jnp.float32(1.0 / f), jnp.float32(0.0))


def _chain_gcn_matrix():
    n = 7 + 1
    r = jax.lax.broadcasted_iota(jnp.int32, (n, n), 0)
    c = jax.lax.broadcasted_iota(jnp.int32, (n, n), 1)
    inv_sqrt2 = 1.0 / jnp.sqrt(jnp.float32(2.0))
    diag = jnp.where(r == c, jnp.where(r == 0, 1.0, 0.5), 0.0)
    sub = jnp.where(r == c + 1, jnp.where(r == 1, inv_sqrt2, 0.5), 0.0)
    return (diag + sub).astype(jnp.float32)


def _fwd_kernel(l_ref, snaps_ref, noise_ref, b1_ref, w2_ref, b2_ref, b3_ref,
                b4_ref, bg1_ref, bg2_ref, bl1_ref, wl2_ref, bl2_ref, wbr_ref,
                bb_ref, bc_ref,
                x_any, w1p_any, w3_any, w4_any, wg1_any, wg2_any, wl1_any,
                wc_any,
                out_h, out_l, out_b, out_p, out_pi,
                xw0, xw1, xw2, w1p_s, w3_s, w4_s, wg1_s, wg2_s, wl1_s, wc_s,
                sems):
    f32 = jnp.float32

    ly = l_ref[0, 0]
    lx = l_ref[0, 1]

    def start(coord, size):
        center = 0.5 * ((coord + 1.0) * IMG)
        n = center.astype(jnp.int32)
        frac = center - n.astype(f32)
        odd = jnp.bitwise_and(n, 1)
        rnd = n + jnp.where(frac > 0.5, 1, jnp.where(frac == 0.5, odd, 0))
        st = rnd - size // 2 + size
        return jnp.clip(st, 0, IMG + size) - size

    xscr = (xw0, xw1, xw2)
    offs = []
    copies = []
    for i, size in enumerate(SIZES):
        d0 = start(ly, size)
        c0 = jnp.clip(d0, 0, IMG - size)
        c0a = pl.multiple_of(jnp.clip((c0 // 8) * 8, 0, IMG - size - 8), 8)
        offs.append(d0 - c0a)
        cp = pltpu.make_async_copy(
            x_any.at[0, pl.ds(c0a, size + 8), :], xscr[i], sems.at[i])
        cp.start()
        copies.append(cp)
    wsrc = (w1p_any, w3_any, w4_any, wg1_any, wg2_any, wl1_any, wc_any)
    wdst = (w1p_s, w3_s, w4_s, wg1_s, wg2_s, wl1_s, wc_s)
    for i in range(len(wsrc)):
        cp = pltpu.make_async_copy(wsrc[i], wdst[i], sems.at[K + i])
        cp.start()
        copies.append(cp)

    r_blk = jax.lax.broadcasted_iota(jnp.int32, (G, G * H_G), 0)
    c_blk = jax.lax.broadcasted_iota(jnp.int32, (G, G * H_G), 1)
    diag_mask = (c_blk // H_G) == r_blk
    j_id = jax.lax.broadcasted_iota(jnp.int32, (G * H_G, H_G), 0)
    o_id = jax.lax.broadcasted_iota(jnp.int32, (G * H_G, H_G), 1)
    block_id = jnp.where(j_id % H_G == o_id, 1.0, 0.0).astype(f32)

    pooled_list = []
    for i, size in enumerate(SIZES):
        f = size // G
        pr = _select_pool_rows(offs[i], f, size + 8)
        pct = _select_pool_cols(start(lx, size), f)
        copies[i].wait()
        pooled = jax.lax.dot(jax.lax.dot(pr, xscr[i][...]), pct)
        pooled_list.append(pooled)

    copies[K].wait()
    g1v = b1_ref[...]
    for i in range(K):
        q = jax.lax.dot(pooled_list[i], w1p_s[i])
        s = jnp.sum(jnp.where(diag_mask, q, 0.0), axis=0, keepdims=True)
        g1v = g1v + jax.lax.dot(s, block_id)
    g1v = jnp.maximum(g1v, 0.0)

    l1 = jnp.maximum(w2_ref[0:1, :] * ly + w2_ref[1:2, :] * lx + b2_ref[...],
                     0.0)

    copies[K + 1].wait()
    copies[K + 2].wait()
    g_t = jnp.maximum(
        (jax.lax.dot(g1v, w3_s[...]) + b3_ref[...])
        + (jax.lax.dot(l1, w4_s[...]) + b4_ref[...]), 0.0)

    nf = jnp.concatenate([snaps_ref[...], g_t], axis=0)
    A = _chain_gcn_matrix()
    copies[K + 3].wait()
    copies[K + 4].wait()
    h1 = jnp.maximum(
        jax.lax.dot(A, jax.lax.dot(nf, wg1_s[...])) + bg1_ref[...], 0.0)
    out2 = jax.lax.dot(A, jax.lax.dot(h1, wg2_s[...])) + bg2_ref[...]
    h_t = jnp.mean(out2, axis=0, keepdims=True)
    out_h[...] = h_t

    copies[K + 5].wait()
    feat = jnp.maximum(jax.lax.dot(h_t, wl1_s[...]) + bl1_ref[...], 0.0)
    mu = jnp.tanh(jax.lax.dot(feat, wl2_ref[...]) + bl2_ref[...])
    l_pre = mu + STD * noise_ref[...]
    out_l[...] = jnp.clip(l_pre, -1.0, 1.0)
    z = (l_pre - mu) / STD
    terms = -0.5 * z * z - jnp.log(f32(STD)) - 0.5 * jnp.log(2.0 * f32(jnp.pi))
    out_pi[...] = jnp.sum(terms, axis=1, keepdims=True)

    out_b[...] = (jnp.sum(h_t * wbr_ref[...], axis=1, keepdims=True)
                  + bb_ref[...])

    copies[K + 6].wait()
    logits = jax.lax.dot(h_t, wc_s[...]) + bc_ref[...]
    m = jnp.max(logits, axis=1, keepdims=True)
    sh = logits - m
    out_p[...] = sh - jnp.log(jnp.sum(jnp.exp(sh), axis=1, keepdims=True))


def kernel(x, l_t_prev, h_t_prev, snaps_prev, noise, params):
    del h_t_prev
    p = params
    f32 = jnp.float32

    xr = x.reshape(x.shape[0], IMG, IMG * C).astype(f32)

    w1p = (p['W1'].reshape(K, G, G * C, H_G)
           .transpose(0, 2, 1, 3)
           .reshape(K, G * C, G * H_G))

    def row(v):
        return v.reshape(1, -1).astype(f32)

    out_shapes = (
        jax.ShapeDtypeStruct((1, HIDDEN), f32),
        jax.ShapeDtypeStruct((1, 2), f32),
        jax.ShapeDtypeStruct((1, 1), f32),
        jax.ShapeDtypeStruct((1, NCLS), f32),
        jax.ShapeDtypeStruct((1, 1), f32),
    )
    n_small = 15
    n_any = 8
    in_specs = ([pl.BlockSpec(memory_space=pltpu.SMEM)] +
                [pl.BlockSpec(memory_space=pltpu.VMEM)
                 for _ in range(n_small)] +
                [pl.BlockSpec(memory_space=pl.ANY) for _ in range(n_any)])
    scratch_shapes = (
        [pltpu.VMEM((sz + 8, IMG * C), f32) for sz in SIZES] +
        [pltpu.VMEM((K, G * C, G * H_G), f32),
         pltpu.VMEM((H_G, HIDDEN), f32),
         pltpu.VMEM((H_G, HIDDEN), f32),
         pltpu.VMEM((HIDDEN, 64), f32),
         pltpu.VMEM((64, HIDDEN), f32),
         pltpu.VMEM((HIDDEN, H_G), f32),
         pltpu.VMEM((HIDDEN, NCLS), f32),
         pltpu.SemaphoreType.DMA((K + 7,))])

    h_t, l_t, b_t, log_probas, log_pi = pl.pallas_call(
        _fwd_kernel,
        out_shape=out_shapes,
        in_specs=in_specs,
        out_specs=tuple(pl.BlockSpec(memory_space=pltpu.VMEM)
                        for _ in range(5)),
        scratch_shapes=scratch_shapes,
    )(l_t_prev[0:1].astype(f32), snaps_prev.astype(f32), noise.astype(f32),
      row(p['b1']), p['W2'], row(p['b2']), row(p['b3']), row(p['b4']),
      row(p['bg1']), row(p['bg2']), row(p['bl1']), p['Wl2'], row(p['bl2']),
      row(p['Wb']), row(p['bb']), row(p['bc']),
      xr, w1p, p['W3'], p['W4'], p['Wg1'], p['Wg2'], p['Wl1'], p['Wc'])

    return (h_t, l_t, b_t.reshape(()), log_probas, log_pi.reshape((1,)))


# speedup vs baseline: 2.7806x; 2.7806x over previous
import jax
import jax.numpy as jnp
from jax.experimental import pallas as pl
from jax.experimental.pallas import tpu as pltpu

G = 16
K = 3
S = 2
C = 3
IMG = 224
H_G = 128
STD = 0.17
HIDDEN = 256
NCLS = 1000
SIZES = tuple(G * (S ** i) for i in range(K))


def _select_pool_rows(off, f, size):
    g = jax.lax.broadcasted_iota(jnp.int32, (G, size), 0)
    u = jax.lax.broadcasted_iota(jnp.int32, (G, size), 1)
    q = u - off - g * f
    sel = jnp.logical_and(q >= 0, q < f)
    return jnp.where(sel, jnp.float32(1.0 / f), jnp.float32(0.0))


def _select_pool_cols(d1, f):
    j = jax.lax.broadcasted_iota(jnp.int32, (IMG * C, G * C), 0)
    t = jax.lax.broadcasted_iota(jnp.int32, (IMG * C, G * C), 1)
    w = j // C
    cj = j - w * C
    g2 = t // C
    ct = t - g2 * C
    q = w - d1 - g2 * f
    sel = jnp.logical_and(jnp.logical_and(q >= 0, q < f), cj == ct)
    return jnp.where(sel, jnp.float32(1.0 / f), jnp.float32(0.0))


def _chain_gcn_matrix():
    n = 7 + 1
    r = jax.lax.broadcasted_iota(jnp.int32, (n, n), 0)
    c = jax.lax.broadcasted_iota(jnp.int32, (n, n), 1)
    inv_sqrt2 = 1.0 / jnp.sqrt(jnp.float32(2.0))
    diag = jnp.where(r == c, jnp.where(r == 0, 1.0, 0.5), 0.0)
    sub = jnp.where(r == c + 1, jnp.where(r == 1, inv_sqrt2, 0.5), 0.0)
    return (diag + sub).astype(jnp.float32)


def _fwd_kernel(l_ref, snaps_ref, noise_ref, b1_ref, w2_ref, b2_ref, b3_ref,
                b4_ref, bg1_ref, bg2_ref, bl1_ref, wl2_ref, bl2_ref, wbr_ref,
                bb_ref, bc_ref,
                x_ref, w1p_any, w3_any, w4_any, wg1_any, wg2_any, wl1_any,
                wc_any,
                out_h, out_l, out_b, out_p, out_pi,
                w1p_s, w3_s, w4_s, wg1_s, wg2_s, wl1_s, wc_s,
                sems):
    f32 = jnp.float32

    ly = l_ref[0, 0]
    lx = l_ref[0, 1]

    def start(coord, size):
        center = 0.5 * ((coord + 1.0) * IMG)
        n = center.astype(jnp.int32)
        frac = center - n.astype(f32)
        odd = jnp.bitwise_and(n, 1)
        rnd = n + jnp.where(frac > 0.5, 1, jnp.where(frac == 0.5, odd, 0))
        st = rnd - size // 2 + size
        return jnp.clip(st, 0, IMG + size) - size

    copies = [None, None, None]
    wsrc = (w1p_any, w3_any, w4_any, wg1_any, wg2_any, wl1_any, wc_any)
    wdst = (w1p_s, w3_s, w4_s, wg1_s, wg2_s, wl1_s, wc_s)
    for i in range(len(wsrc)):
        cp = pltpu.make_async_copy(wsrc[i], wdst[i], sems.at[K + i])
        cp.start()
        copies.append(cp)

    r_blk = jax.lax.broadcasted_iota(jnp.int32, (G, G * H_G), 0)
    c_blk = jax.lax.broadcasted_iota(jnp.int32, (G, G * H_G), 1)
    diag_mask = (c_blk // H_G) == r_blk
    j_id = jax.lax.broadcasted_iota(jnp.int32, (G * H_G, H_G), 0)
    o_id = jax.lax.broadcasted_iota(jnp.int32, (G * H_G, H_G), 1)
    block_id = jnp.where(j_id % H_G == o_id, 1.0, 0.0).astype(f32)

    x2 = x_ref[...]
    pooled_list = []
    for i, size in enumerate(SIZES):
        f = size // G
        pr = _select_pool_rows(start(ly, size), f, IMG)
        pct = _select_pool_cols(start(lx, size), f)
        pooled = jax.lax.dot(jax.lax.dot(pr, x2), pct)
        pooled_list.append(pooled)

    copies[K].wait()
    g1v = b1_ref[...]
    for i in range(K):
        q = jax.lax.dot(pooled_list[i], w1p_s[i])
        s = jnp.sum(jnp.where(diag_mask, q, 0.0), axis=0, keepdims=True)
        g1v = g1v + jax.lax.dot(s, block_id)
    g1v = jnp.maximum(g1v, 0.0)

    l1 = jnp.maximum(w2_ref[0:1, :] * ly + w2_ref[1:2, :] * lx + b2_ref[...],
                     0.0)

    copies[K + 1].wait()
    copies[K + 2].wait()
    g_t = jnp.maximum(
        (jax.lax.dot(g1v, w3_s[...]) + b3_ref[...])
        + (jax.lax.dot(l1, w4_s[...]) + b4_ref[...]), 0.0)

    nf = jnp.concatenate([snaps_ref[...], g_t], axis=0)
    A = _chain_gcn_matrix()
    copies[K + 3].wait()
    copies[K + 4].wait()
    h1 = jnp.maximum(
        jax.lax.dot(A, jax.lax.dot(nf, wg1_s[...])) + bg1_ref[...], 0.0)
    out2 = jax.lax.dot(A, jax.lax.dot(h1, wg2_s[...])) + bg2_ref[...]
    h_t = jnp.mean(out2, axis=0, keepdims=True)
    out_h[...] = h_t

    copies[K + 5].wait()
    feat = jnp.maximum(jax.lax.dot(h_t, wl1_s[...]) + bl1_ref[...], 0.0)
    mu = jnp.tanh(jax.lax.dot(feat, wl2_ref[...]) + bl2_ref[...])
    l_pre = mu + STD * noise_ref[...]
    out_l[...] = jnp.clip(l_pre, -1.0, 1.0)
    z = (l_pre - mu) / STD
    terms = -0.5 * z * z - jnp.log(f32(STD)) - 0.5 * jnp.log(2.0 * f32(jnp.pi))
    out_pi[...] = jnp.sum(terms, axis=1, keepdims=True)

    out_b[...] = (jnp.sum(h_t * wbr_ref[...], axis=1, keepdims=True)
                  + bb_ref[...])

    copies[K + 6].wait()
    logits = jax.lax.dot(h_t, wc_s[...]) + bc_ref[...]
    m = jnp.max(logits, axis=1, keepdims=True)
    sh = logits - m
    out_p[...] = sh - jnp.log(jnp.sum(jnp.exp(sh), axis=1, keepdims=True))


def kernel(x, l_t_prev, h_t_prev, snaps_prev, noise, params):
    del h_t_prev
    p = params
    f32 = jnp.float32

    xr = x[0].reshape(IMG, IMG * C).astype(f32)

    w1p = (p['W1'].reshape(K, G, G * C, H_G)
           .transpose(0, 2, 1, 3)
           .reshape(K, G * C, G * H_G))

    def row(v):
        return v.reshape(1, -1).astype(f32)

    out_shapes = (
        jax.ShapeDtypeStruct((1, HIDDEN), f32),
        jax.ShapeDtypeStruct((1, 2), f32),
        jax.ShapeDtypeStruct((1, 1), f32),
        jax.ShapeDtypeStruct((1, NCLS), f32),
        jax.ShapeDtypeStruct((1, 1), f32),
    )
    n_small = 16
    n_any = 7
    in_specs = ([pl.BlockSpec(memory_space=pltpu.SMEM)] +
                [pl.BlockSpec(memory_space=pltpu.VMEM)
                 for _ in range(n_small)] +
                [pl.BlockSpec(memory_space=pl.ANY) for _ in range(n_any)])
    scratch_shapes = (
        [pltpu.VMEM((K, G * C, G * H_G), f32),
         pltpu.VMEM((H_G, HIDDEN), f32),
         pltpu.VMEM((H_G, HIDDEN), f32),
         pltpu.VMEM((HIDDEN, 64), f32),
         pltpu.VMEM((64, HIDDEN), f32),
         pltpu.VMEM((HIDDEN, H_G), f32),
         pltpu.VMEM((HIDDEN, NCLS), f32),
         pltpu.SemaphoreType.DMA((K + 7,))])

    h_t, l_t, b_t, log_probas, log_pi = pl.pallas_call(
        _fwd_kernel,
        out_shape=out_shapes,
        in_specs=in_specs,
        out_specs=tuple(pl.BlockSpec(memory_space=pltpu.VMEM)
                        for _ in range(5)),
        scratch_shapes=scratch_shapes,
    )(l_t_prev[0:1].astype(f32), snaps_prev.astype(f32), noise.astype(f32),
      row(p['b1']), p['W2'], row(p['b2']), row(p['b3']), row(p['b4']),
      row(p['bg1']), row(p['bg2']), row(p['bl1']), p['Wl2'], row(p['bl2']),
      row(p['Wb']), row(p['bb']), row(p['bc']),
      xr, w1p, p['W3'], p['W4'], p['Wg1'], p['Wg2'], p['Wl1'], p['Wc'])

    return (h_t, l_t, b_t.reshape(()), log_probas, log_pi.reshape((1,)))


# speedup vs baseline: 2.8241x; 1.0156x over previous
import jax
import jax.numpy as jnp
from jax.experimental import pallas as pl
from jax.experimental.pallas import tpu as pltpu

G = 16
K = 3
S = 2
C = 3
IMG = 224
H_G = 128
H_L = 128
STD = 0.17
HIDDEN = 256
NCLS = 1000
PAD = G * (S ** (K - 1))
XP = IMG + 2 * PAD


def _select_pool_rows(d0, f):
    g = jax.lax.broadcasted_iota(jnp.int32, (G, IMG), 0)
    u = jax.lax.broadcasted_iota(jnp.int32, (G, IMG), 1)
    q = u - d0 - g * f
    sel = jnp.logical_and(q >= 0, q < f)
    return jnp.where(sel, jnp.float32(1.0 / f), jnp.float32(0.0))


def _select_pool_cols(d1, f):
    j = jax.lax.broadcasted_iota(jnp.int32, (IMG * C, G * C), 0)
    t = jax.lax.broadcasted_iota(jnp.int32, (IMG * C, G * C), 1)
    w = j // C
    cj = j - w * C
    g2 = t // C
    ct = t - g2 * C
    q = w - d1 - g2 * f
    sel = jnp.logical_and(jnp.logical_and(q >= 0, q < f), cj == ct)
    return jnp.where(sel, jnp.float32(1.0 / f), jnp.float32(0.0))


def _chain_gcn_matrix():
    n = 7 + 1
    r = jax.lax.broadcasted_iota(jnp.int32, (n, n), 0)
    c = jax.lax.broadcasted_iota(jnp.int32, (n, n), 1)
    inv_sqrt2 = 1.0 / jnp.sqrt(jnp.float32(2.0))
    diag = jnp.where(r == c, jnp.where(r == 0, 1.0, 0.5), 0.0)
    sub = jnp.where(r == c + 1, jnp.where(r == 1, inv_sqrt2, 0.5), 0.0)
    return (diag + sub).astype(jnp.float32)


def _fwd_kernel(l_ref, x_ref, snaps_ref, noise_ref, w1p_ref, b1_ref, w2_ref,
                b2_ref, w3_ref, b3_ref, w4_ref, b4_ref, wg1_ref, bg1_ref,
                wg2_ref, bg2_ref, wl1_ref, bl1_ref, wl2_ref, bl2_ref, wb_ref,
                bb_ref, wc_ref, bc_ref,
                out_h, out_l, out_b, out_p, out_pi):
    f32 = jnp.float32

    ly = l_ref[0, 0]
    lx = l_ref[0, 1]

    def start(coord, size):
        center = 0.5 * ((coord + 1.0) * IMG)
        n = center.astype(jnp.int32)
        frac = center - n.astype(f32)
        odd = jnp.bitwise_and(n, 1)
        rnd = n + jnp.where(frac > 0.5, 1, jnp.where(frac == 0.5, odd, 0))
        st = rnd - size // 2 + size
        return jnp.clip(st, 0, IMG + size) - size

    r_blk = jax.lax.broadcasted_iota(jnp.int32, (G, G * H_G), 0)
    c_blk = jax.lax.broadcasted_iota(jnp.int32, (G, G * H_G), 1)
    diag_mask = (c_blk // H_G) == r_blk
    j_id = jax.lax.broadcasted_iota(jnp.int32, (G * H_G, H_G), 0)
    o_id = jax.lax.broadcasted_iota(jnp.int32, (G * H_G, H_G), 1)
    block_id = jnp.where(j_id % H_G == o_id, 1.0, 0.0).astype(f32)

    x2 = x_ref[...]
    g1v = b1_ref[...]
    for i in range(K):
        size = G * (S ** i)
        f = size // G
        d0 = start(ly, size)
        d1 = start(lx, size)
        pr = _select_pool_rows(d0, f)
        pct = _select_pool_cols(d1, f)
        pooled = jax.lax.dot(jax.lax.dot(pr, x2), pct)
        q = jax.lax.dot(pooled, w1p_ref[i])
        s = jnp.sum(jnp.where(diag_mask, q, 0.0), axis=0, keepdims=True)
        g1v = g1v + jax.lax.dot(s, block_id)
    g1v = jnp.maximum(g1v, 0.0)

    l1 = jnp.maximum(w2_ref[0:1, :] * ly + w2_ref[1:2, :] * lx + b2_ref[...],
                     0.0)

    g_t = jnp.maximum(
        (jax.lax.dot(g1v, w3_ref[...]) + b3_ref[...])
        + (jax.lax.dot(l1, w4_ref[...]) + b4_ref[...]), 0.0)

    nf = jnp.concatenate([snaps_ref[...], g_t], axis=0)
    A = _chain_gcn_matrix()
    h1 = jnp.maximum(
        jax.lax.dot(A, jax.lax.dot(nf, wg1_ref[...])) + bg1_ref[...], 0.0)
    out2 = jax.lax.dot(A, jax.lax.dot(h1, wg2_ref[...])) + bg2_ref[...]
    h_t = jnp.mean(out2, axis=0, keepdims=True)
    out_h[...] = h_t

    feat = jnp.maximum(jax.lax.dot(h_t, wl1_ref[...]) + bl1_ref[...], 0.0)
    mu = jnp.tanh(jax.lax.dot(feat, wl2_ref[...]) + bl2_ref[...])
    l_pre = mu + STD * noise_ref[...]
    out_l[...] = jnp.clip(l_pre, -1.0, 1.0)
    z = (l_pre - mu) / STD
    terms = -0.5 * z * z - jnp.log(f32(STD)) - 0.5 * jnp.log(2.0 * f32(jnp.pi))
    out_pi[...] = jnp.sum(terms, axis=1, keepdims=True)

    out_b[...] = jax.lax.dot(h_t, wb_ref[...]) + bb_ref[...]

    logits = jax.lax.dot(h_t, wc_ref[...]) + bc_ref[...]
    m = jnp.max(logits, axis=1, keepdims=True)
    sh = logits - m
    out_p[...] = sh - jnp.log(jnp.sum(jnp.exp(sh), axis=1, keepdims=True))


def kernel(x, l_t_prev, h_t_prev, snaps_prev, noise, params):
    del h_t_prev
    p = params
    f32 = jnp.float32

    def vmem(a):
        return pltpu.with_memory_space_constraint(a, pltpu.MemorySpace.VMEM)

    xr = vmem(x[0].reshape(IMG, IMG * C).astype(f32))

    w1p = vmem(p['W1'].reshape(K, G, G * C, H_G)
               .transpose(0, 2, 1, 3)
               .reshape(K, G * C, G * H_G))

    def row(v):
        return v.reshape(1, -1).astype(f32)

    out_shapes = (
        jax.ShapeDtypeStruct((1, HIDDEN), f32),
        jax.ShapeDtypeStruct((1, 2), f32),
        jax.ShapeDtypeStruct((1, 1), f32),
        jax.ShapeDtypeStruct((1, NCLS), f32),
        jax.ShapeDtypeStruct((1, 1), f32),
    )
    in_specs = ([pl.BlockSpec(memory_space=pltpu.SMEM)] +
                [pl.BlockSpec(memory_space=pltpu.VMEM) for _ in range(23)])

    h_t, l_t, b_t, log_probas, log_pi = pl.pallas_call(
        _fwd_kernel,
        out_shape=out_shapes,
        in_specs=in_specs,
        out_specs=tuple(pl.BlockSpec(memory_space=pltpu.VMEM)
                        for _ in range(5)),
    )(l_t_prev[0:1].astype(f32), xr, snaps_prev.astype(f32),
      noise.astype(f32), w1p,
      row(p['b1']), p['W2'], row(p['b2']), vmem(p['W3']), row(p['b3']),
      vmem(p['W4']), row(p['b4']), vmem(p['Wg1']), row(p['bg1']),
      vmem(p['Wg2']), row(p['bg2']), vmem(p['Wl1']), row(p['bl1']),
      p['Wl2'], row(p['bl2']), p['Wb'], row(p['bb']),
      vmem(p['Wc']), row(p['bc']))

    return (h_t, l_t, b_t.reshape(()), log_probas, log_pi.reshape((1,)))
